# baseline (speedup 1.0000x reference)
import jax
import jax.numpy as jnp
from jax.experimental import pallas as pl
from jax.experimental.pallas import tpu as pltpu

_X_CHUNKS = 16


def _trim_kernel(x_ref, v_ref, m_ref, xo_ref, vo_ref, mo_ref, sem):
    copies = [
        pltpu.make_async_copy(x_ref.at[i], xo_ref.at[i], sem)
        for i in range(_X_CHUNKS)
    ]
    copies.append(pltpu.make_async_copy(v_ref, vo_ref, sem))
    for c in copies:
        c.start()
    mo_ref[...] = m_ref[...] != 0.0
    for c in copies:
        c.wait()


def kernel(x, v, mask):
    xo, vo, mo = pl.pallas_call(
        _trim_kernel,
        in_specs=[
            pl.BlockSpec(memory_space=pl.ANY),
            pl.BlockSpec(memory_space=pl.ANY),
            pl.BlockSpec(memory_space=pltpu.MemorySpace.VMEM),
        ],
        out_specs=[
            pl.BlockSpec(memory_space=pl.ANY),
            pl.BlockSpec(memory_space=pl.ANY),
            pl.BlockSpec(memory_space=pltpu.MemorySpace.VMEM),
        ],
        out_shape=[
            jax.ShapeDtypeStruct(x.shape, x.dtype),
            jax.ShapeDtypeStruct(v.shape, v.dtype),
            jax.ShapeDtypeStruct(mask.shape, jnp.bool_),
        ],
        scratch_shapes=[pltpu.SemaphoreType.DMA],
    )(x, v, mask)
    return (xo, vo, mo)


# speedup vs baseline: 32.9015x; 32.9015x over previous
import jax
import jax.numpy as jnp
from jax.experimental import pallas as pl
from jax.experimental.pallas import tpu as pltpu

_GRID = 8


def _trim_kernel(x_ref, v_ref, m_ref, xo_ref, vo_ref, mo_ref):
    xo_ref[...] = x_ref[...]

    @pl.when(pl.program_id(0) == 0)
    def _():
        vo_ref[...] = v_ref[...]
        mo_ref[...] = m_ref[...] != 0.0


def kernel(x, v, mask):
    b, n, l = x.shape
    _, nv, _ = v.shape
    _, nm, _ = mask.shape
    rows = b * n
    blk = rows // _GRID
    x2 = x.reshape(rows, l)
    xo, vo, mo = pl.pallas_call(
        _trim_kernel,
        grid=(_GRID,),
        in_specs=[
            pl.BlockSpec((blk, l), lambda i: (i, 0)),
            pl.BlockSpec((b, nv, l), lambda i: (0, 0, 0)),
            pl.BlockSpec((b, nm, l), lambda i: (0, 0, 0)),
        ],
        out_specs=[
            pl.BlockSpec((blk, l), lambda i: (i, 0)),
            pl.BlockSpec((b, nv, l), lambda i: (0, 0, 0)),
            pl.BlockSpec((b, nm, l), lambda i: (0, 0, 0)),
        ],
        out_shape=[
            jax.ShapeDtypeStruct((rows, l), x.dtype),
            jax.ShapeDtypeStruct(v.shape, v.dtype),
            jax.ShapeDtypeStruct(mask.shape, jnp.bool_),
        ],
    )(x2, v, mask)
    return (xo.reshape(x.shape), vo, mo)


# speedup vs baseline: 35.0476x; 1.0652x over previous
import jax
import jax.numpy as jnp
from jax.experimental import pallas as pl
from jax.experimental.pallas import tpu as pltpu

_GRID = 4


def _trim_kernel(x_ref, v_ref, m_ref, xo_ref, vo_ref, mo_ref):
    xo_ref[...] = x_ref[...]

    @pl.when(pl.program_id(0) == 0)
    def _():
        vo_ref[...] = v_ref[...]
        mo_ref[...] = m_ref[...] != 0.0


def kernel(x, v, mask):
    b, n, l = x.shape
    _, nv, _ = v.shape
    _, nm, _ = mask.shape
    rows = b * n
    blk = rows // _GRID
    x2 = x.reshape(rows, l)
    xo, vo, mo = pl.pallas_call(
        _trim_kernel,
        grid=(_GRID,),
        in_specs=[
            pl.BlockSpec((blk, l), lambda i: (i, 0)),
            pl.BlockSpec((b, nv, l), lambda i: (0, 0, 0)),
            pl.BlockSpec((b, nm, l), lambda i: (0, 0, 0)),
        ],
        out_specs=[
            pl.BlockSpec((blk, l), lambda i: (i, 0)),
            pl.BlockSpec((b, nv, l), lambda i: (0, 0, 0)),
            pl.BlockSpec((b, nm, l), lambda i: (0, 0, 0)),
        ],
        out_shape=[
            jax.ShapeDtypeStruct((rows, l), x.dtype),
            jax.ShapeDtypeStruct(v.shape, v.dtype),
            jax.ShapeDtypeStruct(mask.shape, jnp.bool_),
        ],
    )(x2, v, mask)
    return (xo.reshape(x.shape), vo, mo)


# speedup vs baseline: 39.3799x; 1.1236x over previous
import jax
import jax.numpy as jnp
from jax.experimental import pallas as pl
from jax.experimental.pallas import tpu as pltpu

_GRID = 2


def _trim_kernel(x_ref, v_ref, m_ref, xo_ref, vo_ref, mo_ref):
    xo_ref[...] = x_ref[...]

    @pl.when(pl.program_id(0) == 0)
    def _():
        vo_ref[...] = v_ref[...]
        mo_ref[...] = m_ref[...] != 0.0


def kernel(x, v, mask):
    b, n, l = x.shape
    _, nv, _ = v.shape
    _, nm, _ = mask.shape
    rows = b * n
    blk = rows // _GRID
    x2 = x.reshape(rows, l)
    xo, vo, mo = pl.pallas_call(
        _trim_kernel,
        grid=(_GRID,),
        in_specs=[
            pl.BlockSpec((blk, l), lambda i: (i, 0)),
            pl.BlockSpec((b, nv, l), lambda i: (0, 0, 0)),
            pl.BlockSpec((b, nm, l), lambda i: (0, 0, 0)),
        ],
        out_specs=[
            pl.BlockSpec((blk, l), lambda i: (i, 0)),
            pl.BlockSpec((b, nv, l), lambda i: (0, 0, 0)),
            pl.BlockSpec((b, nm, l), lambda i: (0, 0, 0)),
        ],
        out_shape=[
            jax.ShapeDtypeStruct((rows, l), x.dtype),
            jax.ShapeDtypeStruct(v.shape, v.dtype),
            jax.ShapeDtypeStruct(mask.shape, jnp.bool_),
        ],
    )(x2, v, mask)
    return (xo.reshape(x.shape), vo, mo)
